# baseline (speedup 1.0000x reference)
import jax
import jax.numpy as jnp
from jax.experimental import pallas as pl
from jax.experimental.pallas import tpu as pltpu


def _fused_body(
    sup_ref, x_ref, wm_ref, wz_ref, b_ref, o_ref, ma_ref, mb_ref, p_ref, acc_ref
):
    s = pl.program_id(0)
    ph = pl.program_id(1)
    i = pl.program_id(2)
    S = pl.num_programs(0)
    RB = sup_ref.shape[1]
    B = x_ref.shape[0]
    Dout = o_ref.shape[2]
    rows = pl.ds(i * RB, RB)

    @pl.when((ph == 0) & (i == 0))
    def _():
        for b in range(B):
            pr = jnp.dot(x_ref[b], wm_ref[0], preferred_element_type=jnp.float32)
            cols = pl.ds(b * Dout, Dout)
            prh = pr.astype(jnp.bfloat16)
            mb_ref[:, cols] = prh[:, 0:Dout]
            ma_ref[:, cols] = prh[:, Dout : 2 * Dout]

    @pl.when(ph == 0)
    def _():
        acc1 = jnp.dot(
            sup_ref[0].astype(jnp.bfloat16),
            ma_ref[...],
            preferred_element_type=jnp.float32,
        )
        p_ref[rows, :] = (acc1 + mb_ref[rows, :].astype(jnp.float32)).astype(
            jnp.bfloat16
        )

    @pl.when(ph == 1)
    def _():
        acc2 = jnp.dot(
            sup_ref[0].astype(jnp.bfloat16),
            p_ref[...],
            preferred_element_type=jnp.float32,
        )

        @pl.when(s == 0)
        def _():
            acc_ref[rows, :] = acc2
            for b in range(B):
                zb = (
                    jnp.dot(
                        x_ref[b, rows, :],
                        wz_ref[...],
                        preferred_element_type=jnp.float32,
                    )
                    + b_ref[...]
                )
                cols = pl.ds(b * Dout, Dout)
                acc_ref[rows, cols] = acc_ref[rows, cols] + zb

        @pl.when(s != 0)
        def _():
            acc_ref[rows, :] = acc_ref[rows, :] + acc2

        @pl.when(s == S - 1)
        def _():
            for b in range(B):
                o_ref[b] = acc_ref[rows, pl.ds(b * Dout, Dout)]


def kernel(supports, inputs, state, output_size, weight, biases):
    S, N, _ = supports.shape
    B = inputs.shape[0]
    Din = inputs.shape[1] // N
    Dh = state.shape[1] // N
    D = Din + Dh
    Dout = weight.shape[1]
    NM = weight.shape[0] // D
    C = B * Dout

    Wr = weight.reshape(D, NM, Dout)
    W0, W1, W2, W3, W4 = (Wr[:, m, :] for m in range(NM))
    Wm = jnp.stack(
        [
            jnp.concatenate([W1, 2.0 * W2], axis=1),
            jnp.concatenate([W3, 2.0 * W4], axis=1),
        ]
    ).astype(jnp.bfloat16)
    Wz = (W0 - W2 - W4).astype(jnp.bfloat16)
    brow = biases.reshape(1, Dout)

    x0 = jnp.concatenate(
        [inputs.reshape(B, N, Din), state.reshape(B, N, Dh)], axis=2
    ).astype(jnp.bfloat16)

    RB = 512
    NBLK = N // RB

    O = pl.pallas_call(
        _fused_body,
        grid=(S, 2, NBLK),
        in_specs=[
            pl.BlockSpec((1, RB, N), lambda s, p, i: (s, i, 0)),
            pl.BlockSpec((B, N, D), lambda s, p, i: (0, 0, 0)),
            pl.BlockSpec((1, D, 2 * Dout), lambda s, p, i: (s, 0, 0)),
            pl.BlockSpec((D, Dout), lambda s, p, i: (0, 0)),
            pl.BlockSpec((1, Dout), lambda s, p, i: (0, 0)),
        ],
        out_specs=pl.BlockSpec((B, RB, Dout), lambda s, p, i: (0, i * p, 0)),
        out_shape=jax.ShapeDtypeStruct((B, N, Dout), jnp.float32),
        scratch_shapes=[
            pltpu.VMEM((N, C), jnp.bfloat16),
            pltpu.VMEM((N, C), jnp.bfloat16),
            pltpu.VMEM((N, C), jnp.bfloat16),
            pltpu.VMEM((N, C), jnp.float32),
        ],
    )(supports, x0, Wm, Wz, brow)

    return O.reshape(B, N * Dout)


# speedup vs baseline: 1.0268x; 1.0268x over previous
import jax
import jax.numpy as jnp
from jax.experimental import pallas as pl
from jax.experimental.pallas import tpu as pltpu


def _proj_body(xi_ref, xs_ref, wi_ref, ws_ref, b_ref, z_ref, mb_ref, ma_ref):
    B = xi_ref.shape[0]
    Dout = z_ref.shape[1] // B
    for b in range(B):
        proj = (
            jnp.dot(xi_ref[b], wi_ref[...], preferred_element_type=jnp.float32)
            + jnp.dot(xs_ref[b], ws_ref[...], preferred_element_type=jnp.float32)
            + b_ref[...]
        )
        cols = pl.ds(b * Dout, Dout)
        projh = proj.astype(jnp.bfloat16)
        z_ref[:, cols] = projh[:, 0:Dout]
        mb_ref[0, :, cols] = projh[:, Dout : 2 * Dout]
        mb_ref[1, :, cols] = projh[:, 2 * Dout : 3 * Dout]
        ma_ref[0, :, cols] = projh[:, 3 * Dout : 4 * Dout]
        ma_ref[1, :, cols] = projh[:, 4 * Dout : 5 * Dout]


def _hop1_body(sup_ref, ma_ref, mb_ref, o_ref):
    acc = jnp.dot(
        sup_ref[0].astype(jnp.bfloat16),
        ma_ref[0],
        preferred_element_type=jnp.float32,
    )
    o_ref[0] = (acc + mb_ref[0].astype(jnp.float32)).astype(jnp.bfloat16)


def _hop2_body(sup_ref, p_ref, z_ref, o_ref, acc_ref):
    s = pl.program_id(1)
    S = pl.num_programs(1)
    B = o_ref.shape[0]
    Dout = o_ref.shape[2]
    acc = jnp.dot(
        sup_ref[0].astype(jnp.bfloat16),
        p_ref[s],
        preferred_element_type=jnp.float32,
    )

    @pl.when(s == 0)
    def _():
        acc_ref[...] = z_ref[...].astype(jnp.float32) + acc

    @pl.when(s != 0)
    def _():
        acc_ref[...] = acc_ref[...] + acc

    @pl.when(s == S - 1)
    def _():
        for b in range(B):
            o_ref[b] = acc_ref[:, pl.ds(b * Dout, Dout)]


def kernel(supports, inputs, state, output_size, weight, biases):
    S, N, _ = supports.shape
    B = inputs.shape[0]
    Din = inputs.shape[1] // N
    Dh = state.shape[1] // N
    D = Din + Dh
    Dout = weight.shape[1]
    NM = weight.shape[0] // D
    C = B * Dout

    Wr = weight.reshape(D, NM, Dout)
    W0, W1, W2, W3, W4 = (Wr[:, m, :] for m in range(NM))
    Wcat = jnp.concatenate(
        [W0 - W2 - W4, W1, W3, 2.0 * W2, 2.0 * W4], axis=1
    )
    Wci = Wcat[:Din]
    Wcs = Wcat[Din:]
    bfull = jnp.concatenate(
        [biases, jnp.zeros((4 * Dout,), biases.dtype)]
    ).reshape(1, 5 * Dout)

    xi = inputs.reshape(B, N, Din)
    xs = state.reshape(B, N, Dh)

    RBn = 512
    z0t, MB, MA = pl.pallas_call(
        _proj_body,
        grid=(N // RBn,),
        in_specs=[
            pl.BlockSpec((B, RBn, Din), lambda i: (0, i, 0)),
            pl.BlockSpec((B, RBn, Dh), lambda i: (0, i, 0)),
            pl.BlockSpec((Din, 5 * Dout), lambda i: (0, 0)),
            pl.BlockSpec((Dh, 5 * Dout), lambda i: (0, 0)),
            pl.BlockSpec((1, 5 * Dout), lambda i: (0, 0)),
        ],
        out_specs=[
            pl.BlockSpec((RBn, C), lambda i: (i, 0)),
            pl.BlockSpec((S, RBn, C), lambda i: (0, i, 0)),
            pl.BlockSpec((S, RBn, C), lambda i: (0, i, 0)),
        ],
        out_shape=[
            jax.ShapeDtypeStruct((N, C), jnp.bfloat16),
            jax.ShapeDtypeStruct((S, N, C), jnp.bfloat16),
            jax.ShapeDtypeStruct((S, N, C), jnp.bfloat16),
        ],
    )(xi, xs, Wci, Wcs, bfull)

    RB = 512

    P = pl.pallas_call(
        _hop1_body,
        grid=(S, N // RB),
        in_specs=[
            pl.BlockSpec((1, RB, N), lambda s, i: (s, i, 0)),
            pl.BlockSpec((1, N, C), lambda s, i: (s, 0, 0)),
            pl.BlockSpec((1, RB, C), lambda s, i: (s, i, 0)),
        ],
        out_specs=pl.BlockSpec((1, RB, C), lambda s, i: (s, i, 0)),
        out_shape=jax.ShapeDtypeStruct((S, N, C), jnp.bfloat16),
    )(supports, MA, MB)

    O = pl.pallas_call(
        _hop2_body,
        grid=(N // RB, S),
        in_specs=[
            pl.BlockSpec((1, RB, N), lambda i, s: (s, i, 0)),
            pl.BlockSpec((S, N, C), lambda i, s: (0, 0, 0)),
            pl.BlockSpec((RB, C), lambda i, s: (i, 0)),
        ],
        out_specs=pl.BlockSpec((B, RB, Dout), lambda i, s: (0, i, 0)),
        out_shape=jax.ShapeDtypeStruct((B, N, Dout), jnp.float32),
        scratch_shapes=[pltpu.VMEM((RB, C), jnp.float32)],
    )(supports, P, z0t)

    return O.reshape(B, N * Dout)
